# baseline (speedup 1.0000x reference)
import functools

import jax
import jax.numpy as jnp
import numpy as np
from jax import lax
from jax.experimental import pallas as pl
from jax.experimental.pallas import tpu as pltpu
from jax.experimental.pallas import tpu_sc as plsc

VOCAB = 1000000
DIM = 64
B = 4096
L = 200
N_SEG = 3

_NC = 2
_NS = 16
_NW = _NC * _NS
_N = B * L
_PW = _N // _NW
_SUB = 128
_NSUB = 8
_CH = _SUB * _NSUB
_NCHUNK = _PW // _CH

_FB = 512


def _sinusoidal_pe(length, dim):
    pos = np.arange(length)[:, None].astype(np.float64)
    i = np.arange(dim)[None, :]
    angle_rates = 1.0 / np.power(10000.0, (2 * (i // 2)) / np.float64(dim))
    angles = pos * angle_rates
    pe = np.zeros((length, dim), dtype=np.float64)
    pe[:, 0::2] = np.sin(angles[:, 0::2])
    pe[:, 1::2] = np.cos(angles[:, 1::2])
    return pe.astype(np.float32)


_PE = _sinusoidal_pe(L, DIM)


def _combo_table(seg_table):
    def body(seg_ref, pe_ref, out_ref):
        out_ref[...] = seg_ref[...] + pe_ref[...]

    out = pl.pallas_call(
        body,
        out_shape=jax.ShapeDtypeStruct((N_SEG, L, DIM), jnp.float32),
    )(seg_table[:, None, :], jnp.asarray(_PE)[None, :, :])
    return out.reshape(N_SEG * L, DIM)


def _sc_lookup(seq_t, lab_t, token_table, combo):
    mesh = plsc.VectorSubcoreMesh(core_axis_name="c", subcore_axis_name="s")

    @functools.partial(
        pl.kernel,
        out_type=jax.ShapeDtypeStruct((_N, DIM), jnp.float32),
        mesh=mesh,
        compiler_params=pltpu.CompilerParams(use_tc_tiling_on_sc=False),
        scratch_types=[
            pltpu.VMEM((_CH,), jnp.int32),
            pltpu.VMEM((_CH,), jnp.int32),
            pltpu.VMEM((_CH,), jnp.int32),
            pltpu.VMEM((_NSUB, _SUB), jnp.int32),
            pltpu.VMEM((_CH, DIM), jnp.float32),
            pltpu.SemaphoreType.DMA,
            pltpu.SemaphoreType.DMA,
            pltpu.SemaphoreType.DMA,
        ],
    )
    def k(seq_hbm, lab_hbm, tok_hbm, combo_hbm, out_hbm,
          idx_v, lab_v, ci_v, oi_v, rows_v, sem_c, sem_t, sem_o):
        wid = lax.axis_index("s") * _NC + lax.axis_index("c")
        lane = lax.iota(jnp.int32, 16)

        def chunk_body(kk, carry):
            s = wid * _PW + kk * _CH
            l = s // B
            b0 = s - l * B
            pltpu.sync_copy(seq_hbm.at[l, pl.ds(b0, _CH)], idx_v)
            pltpu.sync_copy(lab_hbm.at[l, pl.ds(b0, _CH)], lab_v)
            for c in range(_CH // 16):
                ci_v[pl.ds(c * 16, 16)] = lab_v[pl.ds(c * 16, 16)] * L + l
            for j in range(_NSUB):
                for c in range(_SUB // 16):
                    oi_v[j, pl.ds(c * 16, 16)] = (
                        s + 128 * j + 2 * ((c % 4) * 16 + lane) + (c // 4) % 2)
            cps = [pltpu.async_copy(combo_hbm.at[ci_v.at[pl.ds(j * _SUB, _SUB)]],
                                    rows_v.at[pl.ds(j * _SUB, _SUB)], sem_c)
                   for j in range(_NSUB)]
            for cp in cps:
                cp.wait()
            cps = [pltpu.async_copy(tok_hbm.at[idx_v.at[pl.ds(j * _SUB, _SUB)]],
                                    rows_v.at[pl.ds(j * _SUB, _SUB)], sem_t, add=True)
                   for j in range(_NSUB)]
            for cp in cps:
                cp.wait()
            cps = [pltpu.async_copy(rows_v.at[pl.ds(j * _SUB, _SUB)],
                                    out_hbm.at[oi_v.at[j]], sem_o)
                   for j in range(_NSUB)]
            for cp in cps:
                cp.wait()
            return carry

        lax.fori_loop(0, _NCHUNK, chunk_body, 0)

    return k(seq_t, lab_t, token_table, combo)


def _finisher(pairs):
    def body(in_ref, out_ref):
        x = in_ref[...]
        parts = []
        for g in range(_FB // 128):
            sub = x[g * 64:(g + 1) * 64]
            parts.append(sub[:, :DIM].T)
            parts.append(sub[:, DIM:].T)
        out_ref[0] = jnp.concatenate(parts, axis=1)

    nb = B // _FB
    return pl.pallas_call(
        body,
        grid=(L, nb),
        in_specs=[pl.BlockSpec((_FB // 2, 128), lambda l, c: (l * nb + c, 0))],
        out_specs=pl.BlockSpec((1, DIM, _FB), lambda l, c: (l, 0, c)),
        out_shape=jax.ShapeDtypeStruct((L, DIM, B), jnp.float32),
    )(pairs)


def kernel(seq, seg_label, token_table, seg_table):
    combo = _combo_table(seg_table)
    out2d = _sc_lookup(seq.T, seg_label.T, token_table, combo)
    out_t = _finisher(out2d.reshape(_N * DIM // 128, 128))
    return out_t.transpose(2, 0, 1)


# speedup vs baseline: 1.4716x; 1.4716x over previous
import functools

import jax
import jax.numpy as jnp
import numpy as np
from jax import lax
from jax.experimental import pallas as pl
from jax.experimental.pallas import tpu as pltpu
from jax.experimental.pallas import tpu_sc as plsc

VOCAB = 1000000
DIM = 64
B = 4096
L = 200
N_SEG = 3

_NC = 2
_NS = 16
_NW = _NC * _NS
_N = B * L
_PW = _N // _NW
_SUB = 128
_NSUB = 4
_CH = _SUB * _NSUB
_NCHUNK = _PW // _CH

_FB = 2048


def _sinusoidal_pe(length, dim):
    pos = np.arange(length)[:, None].astype(np.float64)
    i = np.arange(dim)[None, :]
    angle_rates = 1.0 / np.power(10000.0, (2 * (i // 2)) / np.float64(dim))
    angles = pos * angle_rates
    pe = np.zeros((length, dim), dtype=np.float64)
    pe[:, 0::2] = np.sin(angles[:, 0::2])
    pe[:, 1::2] = np.cos(angles[:, 1::2])
    return pe.astype(np.float32)


_PE = _sinusoidal_pe(L, DIM)


def _combo_table(seg_table):
    def body(seg_ref, pe_ref, out_ref):
        out_ref[...] = seg_ref[...] + pe_ref[...]

    out = pl.pallas_call(
        body,
        out_shape=jax.ShapeDtypeStruct((N_SEG, L, DIM), jnp.float32),
    )(seg_table[:, None, :], jnp.asarray(_PE)[None, :, :])
    return out.reshape(N_SEG * L, DIM)


def _sc_lookup(seq_t, lab_t, token_table, combo):
    mesh = plsc.VectorSubcoreMesh(core_axis_name="c", subcore_axis_name="s")

    @functools.partial(
        pl.kernel,
        out_type=jax.ShapeDtypeStruct((_N // 2, 2 * DIM), jnp.float32),
        mesh=mesh,
        compiler_params=pltpu.CompilerParams(use_tc_tiling_on_sc=False),
        scratch_types=[
            pltpu.VMEM((2, _CH), jnp.int32),
            pltpu.VMEM((2, _CH), jnp.int32),
            pltpu.VMEM((2, _CH), jnp.int32),
            pltpu.VMEM((2, _CH, DIM), jnp.float32),
            pltpu.SemaphoreType.DMA,
            pltpu.SemaphoreType.DMA,
            pltpu.SemaphoreType.DMA,
            pltpu.SemaphoreType.DMA,
        ],
    )
    def k(seq_hbm, lab_hbm, tok_hbm, combo_hbm, out_hbm,
          idx_v, lab_v, ci_v, rows_v, sem_i, sem_c, sem_t, sem_o):
        wid = lax.axis_index("s") * _NC + lax.axis_index("c")

        def stage(kk):
            s = wid * _PW + kk * _CH
            l = s // B
            b0 = s - l * B
            hb = l * (B // 2) + (b0 // 2048) * 1024 + b0 % 1024
            e = (b0 // 1024) % 2
            return s, l, b0, hb, e

        def load_idx(kk, buf):
            _, l, b0, _, _ = stage(kk)
            pltpu.sync_copy(seq_hbm.at[l, pl.ds(b0, _CH)], idx_v.at[buf])
            pltpu.sync_copy(lab_hbm.at[l, pl.ds(b0, _CH)], lab_v.at[buf])
            for c in range(_CH // 16):
                ci_v[buf, pl.ds(c * 16, 16)] = (
                    lab_v[buf, pl.ds(c * 16, 16)] * L + l)

        def combo_cps(kk, buf):
            return [pltpu.async_copy(
                        combo_hbm.at[ci_v.at[buf, pl.ds(j * _SUB, _SUB)]],
                        rows_v.at[buf, pl.ds(j * _SUB, _SUB)], sem_c)
                    for j in range(_NSUB)]

        def token_cps(kk, buf):
            return [pltpu.async_copy(
                        tok_hbm.at[idx_v.at[buf, pl.ds(j * _SUB, _SUB)]],
                        rows_v.at[buf, pl.ds(j * _SUB, _SUB)], sem_t, add=True)
                    for j in range(_NSUB)]

        def out_dst(kk):
            _, _, _, hb, e = stage(kk)
            return out_hbm.at[pl.ds(hb, _CH), pl.ds(e * DIM, DIM)]

        def out_cp(kk, buf):
            pltpu.async_copy(rows_v.at[buf], out_dst(kk), sem_o)

        def out_wait(kk, buf):
            pltpu.make_async_copy(rows_v.at[buf], out_dst(kk), sem_o).wait()

        load_idx(0, 0)
        for cp in combo_cps(0, 0):
            cp.wait()

        def chunk_body(kk, carry):
            buf = kk % 2
            nbuf = (kk + 1) % 2
            tok = token_cps(kk, buf)
            @pl.when(kk + 1 < _NCHUNK)
            def _prefetch():
                load_idx(kk + 1, nbuf)
                @pl.when(kk >= 1)
                def _drain_prev_write():
                    out_wait(kk - 1, nbuf)
                for cp in combo_cps(kk + 1, nbuf):
                    cp.wait()
            for cp in tok:
                cp.wait()
            out_cp(kk, buf)
            return carry

        lax.fori_loop(0, _NCHUNK, chunk_body, 0)
        out_wait(_NCHUNK - 2, (_NCHUNK - 2) % 2)
        out_wait(_NCHUNK - 1, (_NCHUNK - 1) % 2)

    return k(seq_t, lab_t, token_table, combo)


def _finisher(pairs):
    def body(in_ref, out_ref):
        y = in_ref[...].T
        out_ref[0] = jnp.concatenate([y[:DIM], y[DIM:]], axis=1)

    nb = B // _FB
    return pl.pallas_call(
        body,
        grid=(L, nb),
        in_specs=[pl.BlockSpec((_FB // 2, 2 * DIM), lambda l, c: (l * nb + c, 0))],
        out_specs=pl.BlockSpec((1, DIM, _FB), lambda l, c: (l, 0, c)),
        out_shape=jax.ShapeDtypeStruct((L, DIM, B), jnp.float32),
    )(pairs)


def kernel(seq, seg_label, token_table, seg_table):
    combo = _combo_table(seg_table)
    out_pairs = _sc_lookup(seq.T, seg_label.T, token_table, combo)
    out_t = _finisher(out_pairs)
    return out_t.transpose(2, 0, 1)
